# baseline (speedup 1.0000x reference)
import functools

import jax
import jax.numpy as jnp
from jax.experimental import pallas as pl
from jax.experimental.pallas import tpu as pltpu

B = 64
DIM = 768
HID = 3072
E = 8
HW = 24 * 24


def _stripe_maps():
    r = jax.lax.broadcasted_iota(jnp.int32, (72, 16), 0)
    c = jax.lax.broadcasted_iota(jnp.int32, (72, 16), 1)
    j = r - 9 * (c // 2)
    even = c % 2 == 0
    a = jnp.where(even, ((j >= 0) & (j <= 4)).astype(jnp.float32),
                  ((j >= 5) & (j <= 8)).astype(jnp.float32))
    b = jnp.where(even, ((j >= 0) & (j <= 3)).astype(jnp.float32),
                  ((j >= 4) & (j <= 8)).astype(jnp.float32))
    return a, b


def _pool_kernel(x_ref, o_ref):
    v = x_ref[...]
    rlo = jnp.sum(v[:, :, :64], axis=2)
    rhi = jnp.sum(v[:, :, 64:], axis=2)
    a, b = _stripe_maps()
    s = (jnp.dot(rlo, a, precision=jax.lax.Precision.HIGHEST,
                 preferred_element_type=jnp.float32)
         + jnp.dot(rhi, b, precision=jax.lax.Precision.HIGHEST,
                   preferred_element_type=jnp.float32))
    o_ref[...] = s * (1.0 / HW)


def _moe_kernel(xp_ref, gamma_ref, beta_ref, wg_ref, bg_ref,
                w1_ref, b1_ref, w2_ref, b2_ref,
                ffn_ref, aux_ref,
                xn_ref, coef_ref):
    e = pl.program_id(0)

    @pl.when(e == 0)
    def _gating():
        xp = xp_ref[...]
        mu = jnp.mean(xp, axis=-1, keepdims=True)
        var = jnp.mean((xp - mu) ** 2, axis=-1, keepdims=True)
        xn = (xp - mu) * jax.lax.rsqrt(var + 1e-5) * gamma_ref[...] + beta_ref[...]
        xn_ref[...] = xn
        logits = jax.lax.dot_general(
            xn, wg_ref[...], (((1,), (1,)), ((), ())),
            preferred_element_type=jnp.float32,
            precision=jax.lax.Precision.HIGHEST) + bg_ref[...]
        io = jax.lax.broadcasted_iota(jnp.int32, (B, E), 1)
        v1 = jnp.max(logits, axis=-1, keepdims=True)
        idx1 = jnp.min(jnp.where(logits == v1, io, E), axis=-1, keepdims=True)
        m1 = io == idx1
        logits_m = jnp.where(m1, -jnp.inf, logits)
        v2 = jnp.max(logits_m, axis=-1, keepdims=True)
        idx2 = jnp.min(jnp.where(logits_m == v2, io, E), axis=-1, keepdims=True)
        m2 = io == idx2
        z = jnp.exp(v2 - v1)
        w_a = 1.0 / (1.0 + z)
        w_b = z / (1.0 + z)
        coef_ref[...] = w_a * m1.astype(jnp.float32) + w_b * m2.astype(jnp.float32)
        p = jnp.exp(logits - v1)
        p = p / jnp.sum(p, axis=-1, keepdims=True)
        imp = jnp.sum(p, axis=0, keepdims=True)
        mi = jnp.mean(imp, axis=-1, keepdims=True)
        vi = jnp.mean((imp - mi) ** 2, axis=-1, keepdims=True)
        load = jnp.sum(m1.astype(jnp.float32) + m2.astype(jnp.float32),
                       axis=0, keepdims=True)
        ml = jnp.mean(load, axis=-1, keepdims=True)
        vl = jnp.mean((load - ml) ** 2, axis=-1, keepdims=True)
        aux_ref[...] = vi / (mi * mi + 1e-10) + vl / (ml * ml + 1e-10)
        ffn_ref[...] = jnp.zeros_like(ffn_ref)

    xn = xn_ref[...].astype(jnp.bfloat16)
    h = jax.lax.dot_general(
        xn, w1_ref[0].astype(jnp.bfloat16), (((1,), (0,)), ((), ())),
        preferred_element_type=jnp.float32) + b1_ref[0]
    h = h * jax.nn.sigmoid(h)
    o = jax.lax.dot_general(
        h.astype(jnp.bfloat16), w2_ref[0].astype(jnp.bfloat16),
        (((1,), (0,)), ((), ())),
        preferred_element_type=jnp.float32) + b2_ref[0]
    io = jax.lax.broadcasted_iota(jnp.int32, (B, E), 1)
    c = jnp.sum(jnp.where(io == e, coef_ref[...], 0.0), axis=-1, keepdims=True)
    ffn_ref[...] += c * o


def _add_kernel(x_ref, ffn_ref, o_ref):
    f = ffn_ref[...]
    a, b = _stripe_maps()
    alo = jax.lax.dot_general(
        f, a, (((1,), (1,)), ((), ())),
        precision=jax.lax.Precision.HIGHEST,
        preferred_element_type=jnp.float32)
    ahi = jax.lax.dot_general(
        f, b, (((1,), (1,)), ((), ())),
        precision=jax.lax.Precision.HIGHEST,
        preferred_element_type=jnp.float32)
    v = x_ref[...]
    lane = jax.lax.broadcasted_iota(jnp.int32, v.shape, 2)
    o_ref[...] = v + jnp.where(lane < 64, alo[:, :, None], ahi[:, :, None])


@functools.partial(jax.jit, static_argnames=("interpret",))
def kernel(x, gamma, beta, Wg, bg, W1, b1, W2, b2, interpret=False):
    NG = B * DIM // 16
    GB = 256
    x3 = x.reshape(NG, 72, 128)
    x_pool = pl.pallas_call(
        _pool_kernel,
        grid=(NG // GB,),
        in_specs=[pl.BlockSpec((GB, 72, 128), lambda i: (i, 0, 0))],
        out_specs=pl.BlockSpec((GB, 16), lambda i: (i, 0)),
        out_shape=jax.ShapeDtypeStruct((NG, 16), jnp.float32),
        interpret=interpret,
    )(x3).reshape(B, DIM)

    ffn, aux = pl.pallas_call(
        _moe_kernel,
        grid=(E,),
        in_specs=[
            pl.BlockSpec((B, DIM), lambda e: (0, 0)),
            pl.BlockSpec((1, DIM), lambda e: (0, 0)),
            pl.BlockSpec((1, DIM), lambda e: (0, 0)),
            pl.BlockSpec((E, DIM), lambda e: (0, 0)),
            pl.BlockSpec((1, E), lambda e: (0, 0)),
            pl.BlockSpec((1, DIM, HID), lambda e: (e, 0, 0)),
            pl.BlockSpec((1, 1, HID), lambda e: (e, 0, 0)),
            pl.BlockSpec((1, HID, DIM), lambda e: (e, 0, 0)),
            pl.BlockSpec((1, 1, DIM), lambda e: (e, 0, 0)),
        ],
        out_specs=[
            pl.BlockSpec((B, DIM), lambda e: (0, 0)),
            pl.BlockSpec((1, 1), lambda e: (0, 0)),
        ],
        out_shape=[
            jax.ShapeDtypeStruct((B, DIM), jnp.float32),
            jax.ShapeDtypeStruct((1, 1), jnp.float32),
        ],
        scratch_shapes=[
            pltpu.VMEM((B, DIM), jnp.float32),
            pltpu.VMEM((B, E), jnp.float32),
        ],
        interpret=interpret,
    )(x_pool, gamma.reshape(1, DIM), beta.reshape(1, DIM), Wg,
      bg.reshape(1, E), W1, b1.reshape(E, 1, HID), W2, b2.reshape(E, 1, DIM))

    AB = 128
    out = pl.pallas_call(
        _add_kernel,
        grid=(NG // AB,),
        in_specs=[
            pl.BlockSpec((AB, 72, 128), lambda i: (i, 0, 0)),
            pl.BlockSpec((AB, 16), lambda i: (i, 0)),
        ],
        out_specs=pl.BlockSpec((AB, 72, 128), lambda i: (i, 0, 0)),
        out_shape=jax.ShapeDtypeStruct((NG, 72, 128), jnp.float32),
        interpret=interpret,
    )(x3, ffn.reshape(NG, 16))

    return out.reshape(x.shape), aux[0, 0]


# speedup vs baseline: 12.3285x; 12.3285x over previous
import functools

import jax
import jax.numpy as jnp
from jax.experimental import pallas as pl
from jax.experimental.pallas import tpu as pltpu

B = 64
DIM = 768
HID = 3072
E = 8
HW = 24 * 24


def _pool_kernel(x_ref, o_ref):
    v = x_ref[...]
    pb = o_ref.shape[0]
    s = jnp.sum(v.reshape(pb, HW, DIM), axis=1)
    o_ref[...] = s * (1.0 / HW)


def _moe_kernel(xp_ref, gamma_ref, beta_ref, wg_ref, bg_ref,
                w1_ref, b1_ref, w2_ref, b2_ref,
                ffn_ref, aux_ref,
                xn_ref, coef_ref):
    e = pl.program_id(0)

    @pl.when(e == 0)
    def _gating():
        xp = xp_ref[...]
        mu = jnp.mean(xp, axis=-1, keepdims=True)
        var = jnp.mean((xp - mu) ** 2, axis=-1, keepdims=True)
        xn = (xp - mu) * jax.lax.rsqrt(var + 1e-5) * gamma_ref[...] + beta_ref[...]
        xn_ref[...] = xn
        logits = jax.lax.dot_general(
            xn, wg_ref[...], (((1,), (1,)), ((), ())),
            preferred_element_type=jnp.float32,
            precision=jax.lax.Precision.HIGHEST) + bg_ref[...]
        io = jax.lax.broadcasted_iota(jnp.int32, (B, E), 1)
        v1 = jnp.max(logits, axis=-1, keepdims=True)
        idx1 = jnp.min(jnp.where(logits == v1, io, E), axis=-1, keepdims=True)
        m1 = io == idx1
        logits_m = jnp.where(m1, -jnp.inf, logits)
        v2 = jnp.max(logits_m, axis=-1, keepdims=True)
        idx2 = jnp.min(jnp.where(logits_m == v2, io, E), axis=-1, keepdims=True)
        m2 = io == idx2
        z = jnp.exp(v2 - v1)
        w_a = 1.0 / (1.0 + z)
        w_b = z / (1.0 + z)
        coef_ref[...] = w_a * m1.astype(jnp.float32) + w_b * m2.astype(jnp.float32)
        p = jnp.exp(logits - v1)
        p = p / jnp.sum(p, axis=-1, keepdims=True)
        imp = jnp.sum(p, axis=0, keepdims=True)
        mi = jnp.mean(imp, axis=-1, keepdims=True)
        vi = jnp.mean((imp - mi) ** 2, axis=-1, keepdims=True)
        load = jnp.sum(m1.astype(jnp.float32) + m2.astype(jnp.float32),
                       axis=0, keepdims=True)
        ml = jnp.mean(load, axis=-1, keepdims=True)
        vl = jnp.mean((load - ml) ** 2, axis=-1, keepdims=True)
        aux_ref[...] = vi / (mi * mi + 1e-10) + vl / (ml * ml + 1e-10)
        ffn_ref[...] = jnp.zeros_like(ffn_ref)

    xn = xn_ref[...].astype(jnp.bfloat16)
    h = jax.lax.dot_general(
        xn, w1_ref[0].astype(jnp.bfloat16), (((1,), (0,)), ((), ())),
        preferred_element_type=jnp.float32) + b1_ref[0]
    h = h * jax.nn.sigmoid(h)
    o = jax.lax.dot_general(
        h.astype(jnp.bfloat16), w2_ref[0].astype(jnp.bfloat16),
        (((1,), (0,)), ((), ())),
        preferred_element_type=jnp.float32) + b2_ref[0]
    io = jax.lax.broadcasted_iota(jnp.int32, (B, E), 1)
    c = jnp.sum(jnp.where(io == e, coef_ref[...], 0.0), axis=-1, keepdims=True)
    ffn_ref[...] += c * o


def _add_kernel(x_ref, ffn_ref, o_ref):
    ab = ffn_ref.shape[1]
    v = x_ref[...].reshape(ab, HW, DIM)
    f = ffn_ref[0][:, None, :]
    o_ref[...] = (v + f).reshape(ab * HW, DIM)


@functools.partial(jax.jit, static_argnames=("interpret",))
def kernel(x, gamma, beta, Wg, bg, W1, b1, W2, b2, interpret=False):
    x2 = x.transpose(0, 2, 3, 1).reshape(B * HW, DIM)

    PB = 8
    x_pool = pl.pallas_call(
        _pool_kernel,
        grid=(B // PB,),
        in_specs=[pl.BlockSpec((PB * HW, DIM), lambda i: (i, 0))],
        out_specs=pl.BlockSpec((PB, DIM), lambda i: (i, 0)),
        out_shape=jax.ShapeDtypeStruct((B, DIM), jnp.float32),
        interpret=interpret,
    )(x2)

    ffn, aux = pl.pallas_call(
        _moe_kernel,
        grid=(E,),
        in_specs=[
            pl.BlockSpec((B, DIM), lambda e: (0, 0)),
            pl.BlockSpec((1, DIM), lambda e: (0, 0)),
            pl.BlockSpec((1, DIM), lambda e: (0, 0)),
            pl.BlockSpec((E, DIM), lambda e: (0, 0)),
            pl.BlockSpec((1, E), lambda e: (0, 0)),
            pl.BlockSpec((1, DIM, HID), lambda e: (e, 0, 0)),
            pl.BlockSpec((1, 1, HID), lambda e: (e, 0, 0)),
            pl.BlockSpec((1, HID, DIM), lambda e: (e, 0, 0)),
            pl.BlockSpec((1, 1, DIM), lambda e: (e, 0, 0)),
        ],
        out_specs=[
            pl.BlockSpec((B, DIM), lambda e: (0, 0)),
            pl.BlockSpec((1, 1), lambda e: (0, 0)),
        ],
        out_shape=[
            jax.ShapeDtypeStruct((B, DIM), jnp.float32),
            jax.ShapeDtypeStruct((1, 1), jnp.float32),
        ],
        scratch_shapes=[
            pltpu.VMEM((B, DIM), jnp.float32),
            pltpu.VMEM((B, E), jnp.float32),
        ],
        interpret=interpret,
    )(x_pool, gamma.reshape(1, DIM), beta.reshape(1, DIM), Wg,
      bg.reshape(1, E), W1, b1.reshape(E, 1, HID), W2, b2.reshape(E, 1, DIM))

    AB = 4
    out = pl.pallas_call(
        _add_kernel,
        grid=(B // AB,),
        in_specs=[
            pl.BlockSpec((AB * HW, DIM), lambda i: (i, 0)),
            pl.BlockSpec((1, AB, DIM), lambda i: (i, 0, 0)),
        ],
        out_specs=pl.BlockSpec((AB * HW, DIM), lambda i: (i, 0)),
        out_shape=jax.ShapeDtypeStruct((B * HW, DIM), jnp.float32),
        interpret=interpret,
    )(x2, ffn.reshape(B // AB, AB, DIM))

    out4 = out.reshape(B, 24, 24, DIM).transpose(0, 3, 1, 2)
    return out4, aux[0, 0]


# speedup vs baseline: 12.3460x; 1.0014x over previous
import functools

import jax
import jax.numpy as jnp
from jax.experimental import pallas as pl
from jax.experimental.pallas import tpu as pltpu

B = 64
DIM = 768
HID = 3072
E = 8
HW = 24 * 24

PBAT = 2
NP = B // PBAT
NA = B // PBAT


def _fused_kernel(x_ref, gamma_ref, beta_ref, wg_ref, bg_ref,
                  w1_ref, b1_ref, w2_ref, b2_ref,
                  o_ref, aux_ref,
                  xp_ref, xn_ref, coef_ref, ffn_ref):
    i = pl.program_id(0)

    @pl.when(i < NP)
    def _pool():
        v = x_ref[...].reshape(PBAT, HW, DIM)
        xp_ref[i] = jnp.sum(v, axis=1) * (1.0 / HW)

    @pl.when(i == NP)
    def _gating():
        xp = xp_ref[...].reshape(B, DIM)
        mu = jnp.mean(xp, axis=-1, keepdims=True)
        var = jnp.mean((xp - mu) ** 2, axis=-1, keepdims=True)
        xn = (xp - mu) * jax.lax.rsqrt(var + 1e-5) * gamma_ref[...] + beta_ref[...]
        xn_ref[...] = xn
        logits = jax.lax.dot_general(
            xn, wg_ref[...], (((1,), (1,)), ((), ())),
            preferred_element_type=jnp.float32,
            precision=jax.lax.Precision.HIGHEST) + bg_ref[...]
        io = jax.lax.broadcasted_iota(jnp.int32, (B, E), 1)
        v1 = jnp.max(logits, axis=-1, keepdims=True)
        idx1 = jnp.min(jnp.where(logits == v1, io, E), axis=-1, keepdims=True)
        m1 = io == idx1
        logits_m = jnp.where(m1, -jnp.inf, logits)
        v2 = jnp.max(logits_m, axis=-1, keepdims=True)
        idx2 = jnp.min(jnp.where(logits_m == v2, io, E), axis=-1, keepdims=True)
        m2 = io == idx2
        z = jnp.exp(v2 - v1)
        w_a = 1.0 / (1.0 + z)
        w_b = z / (1.0 + z)
        coef_ref[...] = w_a * m1.astype(jnp.float32) + w_b * m2.astype(jnp.float32)
        p = jnp.exp(logits - v1)
        p = p / jnp.sum(p, axis=-1, keepdims=True)
        imp = jnp.sum(p, axis=0, keepdims=True)
        mi = jnp.mean(imp, axis=-1, keepdims=True)
        vi = jnp.mean((imp - mi) ** 2, axis=-1, keepdims=True)
        load = jnp.sum(m1.astype(jnp.float32) + m2.astype(jnp.float32),
                       axis=0, keepdims=True)
        ml = jnp.mean(load, axis=-1, keepdims=True)
        vl = jnp.mean((load - ml) ** 2, axis=-1, keepdims=True)
        aux_ref[...] = vi / (mi * mi + 1e-10) + vl / (ml * ml + 1e-10)
        ffn_ref[...] = jnp.zeros_like(ffn_ref)

    @pl.when((i >= NP) & (i < NP + E))
    def _expert():
        e = i - NP
        xn = xn_ref[...].astype(jnp.bfloat16)
        h = jax.lax.dot_general(
            xn, w1_ref[0].astype(jnp.bfloat16), (((1,), (0,)), ((), ())),
            preferred_element_type=jnp.float32) + b1_ref[0]
        h = h * jax.nn.sigmoid(h)
        o = jax.lax.dot_general(
            h.astype(jnp.bfloat16), w2_ref[0].astype(jnp.bfloat16),
            (((1,), (0,)), ((), ())),
            preferred_element_type=jnp.float32) + b2_ref[0]
        io = jax.lax.broadcasted_iota(jnp.int32, (B, E), 1)
        c = jnp.sum(jnp.where(io == e, coef_ref[...], 0.0), axis=-1,
                    keepdims=True)
        ffn_ref[...] += (c * o).reshape(NA, PBAT, DIM)

    @pl.when(i >= NP + E)
    def _add():
        f = ffn_ref[i - NP - E]
        v = x_ref[...].reshape(PBAT, HW, DIM)
        o_ref[...] = (v + f[:, None, :]).reshape(PBAT * HW, DIM)


@functools.partial(jax.jit, static_argnames=("interpret",))
def kernel(x, gamma, beta, Wg, bg, W1, b1, W2, b2, interpret=False):
    x2 = x.transpose(0, 2, 3, 1).reshape(B * HW, DIM)

    def im_x(i):
        return (jnp.where(i < NP, i, jnp.where(i < NP + E, NP - 1, i - NP - E)),
                0)

    def im_w(i):
        return (jnp.clip(i - NP, 0, E - 1), 0, 0)

    def im_out(i):
        return (jnp.where(i < NP + E, 0, i - NP - E), 0)

    out, aux = pl.pallas_call(
        _fused_kernel,
        grid=(NP + E + NA,),
        in_specs=[
            pl.BlockSpec((PBAT * HW, DIM), im_x),
            pl.BlockSpec((1, DIM), lambda i: (0, 0)),
            pl.BlockSpec((1, DIM), lambda i: (0, 0)),
            pl.BlockSpec((E, DIM), lambda i: (0, 0)),
            pl.BlockSpec((1, E), lambda i: (0, 0)),
            pl.BlockSpec((1, DIM, HID), im_w),
            pl.BlockSpec((1, 1, HID), im_w),
            pl.BlockSpec((1, HID, DIM), im_w),
            pl.BlockSpec((1, 1, DIM), im_w),
        ],
        out_specs=[
            pl.BlockSpec((PBAT * HW, DIM), im_out),
            pl.BlockSpec((1, 1), lambda i: (0, 0)),
        ],
        out_shape=[
            jax.ShapeDtypeStruct((B * HW, DIM), jnp.float32),
            jax.ShapeDtypeStruct((1, 1), jnp.float32),
        ],
        scratch_shapes=[
            pltpu.VMEM((NP, PBAT, DIM), jnp.float32),
            pltpu.VMEM((B, DIM), jnp.float32),
            pltpu.VMEM((B, E), jnp.float32),
            pltpu.VMEM((NA, PBAT, DIM), jnp.float32),
        ],
        interpret=interpret,
    )(x2, gamma.reshape(1, DIM), beta.reshape(1, DIM), Wg,
      bg.reshape(1, E), W1, b1.reshape(E, 1, HID), W2, b2.reshape(E, 1, DIM))

    out4 = out.reshape(B, 24, 24, DIM).transpose(0, 3, 1, 2)
    return out4, aux[0, 0]
